# Initial kernel scaffold; baseline (speedup 1.0000x reference)
import jax
import jax.numpy as jnp
from jax.experimental import pallas as pl


def kernel(x_bus, edge_index_ac, edge_attr_ac, edge_index_tr, edge_attr_tr, W_enc_node, b_enc_node, W_enc_ac, b_enc_ac, W_enc_tr, b_enc_tr, W_msg_ac, b_msg_ac, W_msg_tr, b_msg_tr, W_upd, b_upd, W_dec, b_dec):
    raise NotImplementedError("write your pallas kernel here")


# speedup vs baseline: 2.9819x; 2.9819x over previous
import functools

import jax
import jax.numpy as jnp
from jax import lax
from jax.experimental import pallas as pl
from jax.experimental.pallas import tpu as pltpu
from jax.experimental.pallas import tpu_sc as plsc

N = 10000
EAC = 320000
ETR = 32000
H = 128
K_STEPS = 4
LANES = 16
NC, NS = 2, 16
NW = NC * NS

AC_PER_W = EAC // NW
TR_PER_W = ETR // NW
AC_CH = 80
TR_CH = 40
AC_NCH = AC_PER_W // AC_CH
TR_NCH = TR_PER_W // TR_CH

ECAT = EAC + ETR
PP_AC_CH = 80
PP_TR_CH = 40


@functools.cache
def _sc_mesh():
    return plsc.VectorSubcoreMesh(core_axis_name="c", subcore_axis_name="s",
                                  num_cores=NC, num_subcores=NS)


def _f32(*shape):
    return jax.ShapeDtypeStruct(shape, jnp.float32)


def _mlp_relu(x, W, b, blk):
    M, K = x.shape
    Hd = W.shape[1]

    def body(xr, wr, br, outr):
        outr[...] = jnp.maximum(
            jnp.dot(xr[...], wr[...], preferred_element_type=jnp.float32)
            + br[...], 0.0)

    return pl.pallas_call(
        body,
        grid=(M // blk,),
        in_specs=[pl.BlockSpec((blk, K), lambda i: (i, 0)),
                  pl.BlockSpec((K, Hd), lambda i: (0, 0)),
                  pl.BlockSpec((1, Hd), lambda i: (0, 0))],
        out_specs=pl.BlockSpec((blk, Hd), lambda i: (i, 0)),
        out_shape=_f32(M, Hd),
    )(x, W, b)


def _msg_first(s_rows, d_rows, e0, Wsd, We, b, blk):
    M = e0.shape[0]

    def body(sr, dr, er, wsdr, wer, br, mout):
        sd = jnp.concatenate([sr[...], dr[...]], axis=1)
        d1 = jnp.dot(sd, wsdr[...], preferred_element_type=jnp.float32)
        d3 = jnp.dot(er[...], wer[...], preferred_element_type=jnp.float32)
        mout[...] = jnp.maximum(d1 + d3 + br[...], 0.0)

    sp = pl.BlockSpec((blk, H), lambda i: (i, 0))
    return pl.pallas_call(
        body,
        grid=(M // blk,),
        in_specs=[sp, sp, sp,
                  pl.BlockSpec((2 * H, H), lambda i: (0, 0)),
                  pl.BlockSpec((H, H), lambda i: (0, 0)),
                  pl.BlockSpec((1, H), lambda i: (0, 0))],
        out_specs=sp,
        out_shape=_f32(M, H),
    )(s_rows, d_rows, e0, Wsd, We, b)


def _msg_step(s_rows, d_rows, e_prev, m_prev, Wsd, We, b, blk):
    M = e_prev.shape[0]

    def body(sr, dr, er, mr, wsdr, wer, br, enew, mout):
        e = er[...] + mr[...]
        enew[...] = e
        sd = jnp.concatenate([sr[...], dr[...]], axis=1)
        d1 = jnp.dot(sd, wsdr[...], preferred_element_type=jnp.float32)
        d3 = jnp.dot(e, wer[...], preferred_element_type=jnp.float32)
        mout[...] = jnp.maximum(d1 + d3 + br[...], 0.0)

    sp = pl.BlockSpec((blk, H), lambda i: (i, 0))
    return pl.pallas_call(
        body,
        grid=(M // blk,),
        in_specs=[sp, sp, sp, sp,
                  pl.BlockSpec((2 * H, H), lambda i: (0, 0)),
                  pl.BlockSpec((H, H), lambda i: (0, 0)),
                  pl.BlockSpec((1, H), lambda i: (0, 0))],
        out_specs=[sp, sp],
        out_shape=(_f32(M, H), _f32(M, H)),
    )(s_rows, d_rows, e_prev, m_prev, Wsd, We, b)


def _node_update(nodes, agg, Wu, bu, blk):
    def body(nr, ar, wr, br, outr):
        nb = nr[...]
        na = jnp.concatenate([nb, ar[...]], axis=1)
        upd = jnp.maximum(
            jnp.dot(na, wr[...], preferred_element_type=jnp.float32)
            + br[...], 0.0)
        outr[...] = nb + upd

    sp = pl.BlockSpec((blk, H), lambda i: (i, 0))
    return pl.pallas_call(
        body,
        grid=(N // blk,),
        in_specs=[sp, sp,
                  pl.BlockSpec((2 * H, H), lambda i: (0, 0)),
                  pl.BlockSpec((1, H), lambda i: (0, 0))],
        out_specs=sp,
        out_shape=_f32(N, H),
    )(nodes, agg, Wu, bu)


def _decoder(nodes, Wd8, bd8):
    def body(nr, wr, br, outr):
        outr[...] = jnp.dot(nr[...], wr[...],
                            preferred_element_type=jnp.float32) + br[...]

    return pl.pallas_call(
        body,
        grid=(5,),
        in_specs=[pl.BlockSpec((2000, H), lambda i: (i, 0)),
                  pl.BlockSpec((H, 8), lambda i: (0, 0)),
                  pl.BlockSpec((1, 8), lambda i: (0, 0))],
        out_specs=pl.BlockSpec((2000, 8), lambda i: (i, 0)),
        out_shape=_f32(N, 8),
    )(nodes, Wd8, bd8)


def _postproc(params, vai_a, vmi_a, vaj_a, vmj_a):
    rows = params[0].shape[0]
    has_tap = len(params) == 6
    nin = len(params) + 4

    def body(*refs):
        if has_tap:
            r, x, bfr, bto, tap, shift = (refs[k][...] for k in range(6))
        else:
            r, x, bfr, bto = (refs[k][...] for k in range(4))
        vai, vmi, vaj, vmj = (refs[len(params) + k][...] for k in range(4))
        o_refr, o_imfr, o_reto, o_imto = refs[nin:]
        den = r * r + x * x
        g = r / den
        by = -x / den
        if has_tap:
            th = vai - vaj - shift
            w = vmi * vmj / tap
            p2 = vmi * vmi / (tap * tap)
        else:
            th = vai - vaj
            w = vmi * vmj
            p2 = vmi * vmi
        c = jnp.cos(th)
        s = jnp.sin(th)
        q2 = vmj * vmj
        gc = g * c
        bys = by * s
        gs = g * s
        byc = by * c
        o_refr[...] = g * p2 - w * (gc + bys)
        o_imfr[...] = -(by + bfr) * p2 - w * (gs - byc)
        o_reto[...] = g * q2 - w * (gc - bys)
        o_imto[...] = -(by + bto) * q2 + w * (gs + byc)

    shp = _f32(rows, 128)
    return pl.pallas_call(
        body,
        out_shape=(shp, shp, shp, shp),
    )(*params, vai_a, vmi_a, vaj_a, vmj_a)


def _sc_rows_body(nodes, sac, dac, s_tr, d_tr,
                  sr_ac, dr_ac, sr_tr, dr_tr,
                  sidx, didx, rA, rB, tsidx, tdidx, nsh, semA, semB):
    c = lax.axis_index("c")
    s = lax.axis_index("s")
    wid = c * NS + s

    base0 = s * 624
    pltpu.sync_copy(nodes.at[pl.ds(base0, 624)], nsh.at[pl.ds(base0, 624)])

    @pl.when(s == NS - 1)
    def _stage_tail():
        pltpu.sync_copy(nodes.at[pl.ds(NS * 624, 16)],
                        nsh.at[pl.ds(NS * 624, 16)])
    plsc.subcore_barrier()

    def make_chunk(src, dst, outS, outD, per_w, ch, bsidx, bdidx):
        bA = rA.at[pl.ds(0, ch)]
        bB = rB.at[pl.ds(0, ch)]

        def chunk(j, carry):
            base = wid * per_w + j * ch
            pltpu.sync_copy(src.at[pl.ds(base, ch)], bsidx)
            pltpu.sync_copy(dst.at[pl.ds(base, ch)], bdidx)
            cpA = pltpu.async_copy(nsh.at[bsidx], bA, semA)
            cpB = pltpu.async_copy(nsh.at[bdidx], bB, semB)
            cpA.wait()
            cpB.wait()
            pltpu.sync_copy(bA, outS.at[pl.ds(base, ch)])
            pltpu.sync_copy(bB, outD.at[pl.ds(base, ch)])
            return carry
        return chunk

    lax.fori_loop(0, AC_NCH,
                  make_chunk(sac, dac, sr_ac, dr_ac, AC_PER_W, AC_CH,
                             sidx, didx), 0)
    lax.fori_loop(0, TR_NCH,
                  make_chunk(s_tr, d_tr, sr_tr, dr_tr, TR_PER_W, TR_CH,
                             tsidx, tdidx), 0)


@functools.cache
def _sc_rows_kernel():
    return pl.kernel(
        _sc_rows_body,
        out_type=(_f32(EAC, H), _f32(EAC, H), _f32(ETR, H), _f32(ETR, H)),
        mesh=_sc_mesh(),
        scratch_types=[
            pltpu.VMEM((AC_CH,), jnp.int32),
            pltpu.VMEM((AC_CH,), jnp.int32),
            pltpu.VMEM((AC_CH, H), jnp.float32),
            pltpu.VMEM((AC_CH, H), jnp.float32),
            pltpu.VMEM((TR_CH,), jnp.int32),
            pltpu.VMEM((TR_CH,), jnp.int32),
            pltpu.VMEM_SHARED((N, H), jnp.float32),
            pltpu.SemaphoreType.DMA,
            pltpu.SemaphoreType.DMA,
        ],
    )


def _sc_rows(*args):
    return _sc_rows_kernel()(*args)


def _sc_gather_body(tab, icat, jcat, vai_h, vmi_h, vaj_h, vmj_h,
                    tabv, sidx, didx, tsidx, tdidx, bai, bmi, baj, bmj):
    c = lax.axis_index("c")
    s = lax.axis_index("s")
    wid = c * NS + s

    pltpu.sync_copy(tab, tabv)
    zero16 = jnp.zeros((LANES,), jnp.int32)
    tsidx[pl.ds(32, LANES)] = zero16
    tdidx[pl.ds(32, LANES)] = zero16

    def gather16(bsidx, bdidx, k):
        sl = pl.ds(k * LANES, LANES)
        sr = bsidx[sl] * 8
        dr = bdidx[sl] * 8
        bai[sl] = plsc.load_gather(tabv, [sr])
        bmi[sl] = plsc.load_gather(tabv, [sr + 3])
        baj[sl] = plsc.load_gather(tabv, [dr])
        bmj[sl] = plsc.load_gather(tabv, [dr + 3])

    def ac_chunk(j, carry):
        base = wid * AC_PER_W + j * PP_AC_CH
        pltpu.sync_copy(icat.at[pl.ds(base, PP_AC_CH)], sidx)
        pltpu.sync_copy(jcat.at[pl.ds(base, PP_AC_CH)], didx)
        for k in range(PP_AC_CH // LANES):
            gather16(sidx, didx, k)
        pltpu.sync_copy(bai.at[pl.ds(0, PP_AC_CH)], vai_h.at[pl.ds(base, PP_AC_CH)])
        pltpu.sync_copy(bmi.at[pl.ds(0, PP_AC_CH)], vmi_h.at[pl.ds(base, PP_AC_CH)])
        pltpu.sync_copy(baj.at[pl.ds(0, PP_AC_CH)], vaj_h.at[pl.ds(base, PP_AC_CH)])
        pltpu.sync_copy(bmj.at[pl.ds(0, PP_AC_CH)], vmj_h.at[pl.ds(base, PP_AC_CH)])
        return carry
    lax.fori_loop(0, AC_PER_W // PP_AC_CH, ac_chunk, 0)

    def tr_chunk(j, carry):
        base = EAC + wid * TR_PER_W + j * PP_TR_CH
        pltpu.sync_copy(icat.at[pl.ds(base, PP_TR_CH)], tsidx.at[pl.ds(0, PP_TR_CH)])
        pltpu.sync_copy(jcat.at[pl.ds(base, PP_TR_CH)], tdidx.at[pl.ds(0, PP_TR_CH)])
        for k in range(3):
            gather16(tsidx, tdidx, k)
        pltpu.sync_copy(bai.at[pl.ds(0, PP_TR_CH)], vai_h.at[pl.ds(base, PP_TR_CH)])
        pltpu.sync_copy(bmi.at[pl.ds(0, PP_TR_CH)], vmi_h.at[pl.ds(base, PP_TR_CH)])
        pltpu.sync_copy(baj.at[pl.ds(0, PP_TR_CH)], vaj_h.at[pl.ds(base, PP_TR_CH)])
        pltpu.sync_copy(bmj.at[pl.ds(0, PP_TR_CH)], vmj_h.at[pl.ds(base, PP_TR_CH)])
        return carry
    lax.fori_loop(0, TR_PER_W // PP_TR_CH, tr_chunk, 0)


@functools.cache
def _sc_gather_kernel():
    return pl.kernel(
        _sc_gather_body,
        out_type=(_f32(ECAT), _f32(ECAT), _f32(ECAT), _f32(ECAT)),
        mesh=_sc_mesh(),
        compiler_params=pltpu.CompilerParams(needs_layout_passes=False),
        scratch_types=[
            pltpu.VMEM((N * 8,), jnp.float32),
            pltpu.VMEM((PP_AC_CH,), jnp.int32),
            pltpu.VMEM((PP_AC_CH,), jnp.int32),
            pltpu.VMEM((48,), jnp.int32),
            pltpu.VMEM((48,), jnp.int32),
            pltpu.VMEM((PP_AC_CH,), jnp.float32),
            pltpu.VMEM((PP_AC_CH,), jnp.float32),
            pltpu.VMEM((PP_AC_CH,), jnp.float32),
            pltpu.VMEM((PP_AC_CH,), jnp.float32),
        ],
    )


def _sc_gather(*args):
    return _sc_gather_kernel()(*args)


@jax.jit
def kernel(x_bus, edge_index_ac, edge_attr_ac, edge_index_tr, edge_attr_tr,
           W_enc_node, b_enc_node, W_enc_ac, b_enc_ac, W_enc_tr, b_enc_tr,
           W_msg_ac, b_msg_ac, W_msg_tr, b_msg_tr, W_upd, b_upd,
           W_dec, b_dec):
    sac = edge_index_ac[0]
    dac = edge_index_ac[1]
    s_tr = edge_index_tr[0]
    d_tr = edge_index_tr[1]

    xb8 = jnp.pad(x_bus, ((0, 0), (0, 4)))
    Wn8 = jnp.pad(W_enc_node, ((0, 4), (0, 0)))
    attr_ac16 = jnp.pad(edge_attr_ac, ((0, 0), (0, 7)))
    Wac16 = jnp.pad(W_enc_ac, ((0, 7), (0, 0)))
    attr_tr16 = jnp.pad(edge_attr_tr, ((0, 0), (0, 5)))
    Wtr16 = jnp.pad(W_enc_tr, ((0, 5), (0, 0)))
    Wd8 = jnp.pad(W_dec, ((0, 0), (0, 4)))
    bd8 = jnp.pad(b_dec, (0, 4)).reshape(1, 8)

    nodes = _mlp_relu(xb8, Wn8, b_enc_node.reshape(1, H), 2000)
    e_ac = _mlp_relu(attr_ac16, Wac16, b_enc_ac.reshape(1, H), 8000)
    e_tr = _mlp_relu(attr_tr16, Wtr16, b_enc_tr.reshape(1, H), 8000)

    m_ac = None
    m_tr = None
    for l in range(K_STEPS):
        Wsd_ac = W_msg_ac[l][:2 * H]
        We_ac = W_msg_ac[l][2 * H:]
        Wsd_tr = W_msg_tr[l][:2 * H]
        We_tr = W_msg_tr[l][2 * H:]
        bm_ac = b_msg_ac[l].reshape(1, H)
        bm_tr = b_msg_tr[l].reshape(1, H)

        sr_ac, dr_ac, sr_tr, dr_tr = _sc_rows(nodes, sac, dac, s_tr, d_tr)

        if l == 0:
            m_ac = _msg_first(sr_ac, dr_ac, e_ac, Wsd_ac, We_ac, bm_ac, 8000)
            m_tr = _msg_first(sr_tr, dr_tr, e_tr, Wsd_tr, We_tr, bm_tr, 8000)
        else:
            e_ac, m_ac = _msg_step(sr_ac, dr_ac, e_ac, m_ac,
                                   Wsd_ac, We_ac, bm_ac, 8000)
            e_tr, m_tr = _msg_step(sr_tr, dr_tr, e_tr, m_tr,
                                   Wsd_tr, We_tr, bm_tr, 8000)

        agg = jnp.zeros((N, H), jnp.float32).at[dac].add(m_ac)
        agg = agg.at[d_tr].add(m_tr)

        nodes = _node_update(nodes, agg, W_upd[l], b_upd[l].reshape(1, H),
                             2000)

    out8 = _decoder(nodes, Wd8, bd8)
    out_bus = out8[:, :4]

    icat = jnp.concatenate([sac, s_tr])
    jcat = jnp.concatenate([dac, d_tr])
    vai, vmi, vaj, vmj = _sc_gather(out8.reshape(-1), icat, jcat)

    def _rs(v, lo, hi):
        return v[lo:hi].reshape(-1, 128)

    ac_params = tuple(_rs(edge_attr_ac[:, k], 0, EAC) for k in (4, 5, 2, 3))
    tr_params = tuple(_rs(edge_attr_tr[:, k], 0, ETR)
                      for k in (2, 3, 9, 10, 7, 8))
    pp_ac = _postproc(ac_params, _rs(vai, 0, EAC), _rs(vmi, 0, EAC),
                      _rs(vaj, 0, EAC), _rs(vmj, 0, EAC))
    pp_tr = _postproc(tr_params, _rs(vai, EAC, ECAT), _rs(vmi, EAC, ECAT),
                      _rs(vaj, EAC, ECAT), _rs(vmj, EAC, ECAT))

    re_fr = jnp.concatenate([pp_ac[0].reshape(-1), pp_tr[0].reshape(-1)])
    im_fr = jnp.concatenate([pp_ac[1].reshape(-1), pp_tr[1].reshape(-1)])
    re_to = jnp.concatenate([pp_ac[2].reshape(-1), pp_tr[2].reshape(-1)])
    im_to = jnp.concatenate([pp_ac[3].reshape(-1), pp_tr[3].reshape(-1)])
    return (out_bus, re_fr, im_fr, re_to, im_to)


# speedup vs baseline: 3.2140x; 1.0778x over previous
import functools

import jax
import jax.numpy as jnp
from jax import lax
from jax.experimental import pallas as pl
from jax.experimental.pallas import tpu as pltpu
from jax.experimental.pallas import tpu_sc as plsc

N = 10000
EAC = 320000
ETR = 32000
H = 128
K_STEPS = 4
LANES = 16
NC, NS = 2, 16
NW = NC * NS

AC_PER_W = EAC // NW
TR_PER_W = ETR // NW
AC_CH = 80
TR_CH = 40
AC_NCH = AC_PER_W // AC_CH
TR_NCH = TR_PER_W // TR_CH

ECAT = EAC + ETR
PP_AC_CH = 80
PP_TR_CH = 40


@functools.cache
def _sc_mesh():
    return plsc.VectorSubcoreMesh(core_axis_name="c", subcore_axis_name="s",
                                  num_cores=NC, num_subcores=NS)


def _f32(*shape):
    return jax.ShapeDtypeStruct(shape, jnp.float32)


def _mlp_relu(x, W, b, blk):
    M, K = x.shape
    Hd = W.shape[1]

    def body(xr, wr, br, outr):
        outr[...] = jnp.maximum(
            jnp.dot(xr[...], wr[...], preferred_element_type=jnp.float32)
            + br[...], 0.0)

    return pl.pallas_call(
        body,
        grid=(M // blk,),
        in_specs=[pl.BlockSpec((blk, K), lambda i: (i, 0)),
                  pl.BlockSpec((K, Hd), lambda i: (0, 0)),
                  pl.BlockSpec((1, Hd), lambda i: (0, 0))],
        out_specs=pl.BlockSpec((blk, Hd), lambda i: (i, 0)),
        out_shape=_f32(M, Hd),
    )(x, W, b)


def _msg_first(s_rows, d_rows, e0, Wsd, We, b, blk):
    M = e0.shape[0]

    def body(sr, dr, er, wsdr, wer, br, mout):
        sd = jnp.concatenate([sr[...], dr[...]], axis=1)
        d1 = jnp.dot(sd, wsdr[...], preferred_element_type=jnp.float32)
        d3 = jnp.dot(er[...], wer[...], preferred_element_type=jnp.float32)
        mout[...] = jnp.maximum(d1 + d3 + br[...], 0.0)

    sp = pl.BlockSpec((blk, H), lambda i: (i, 0))
    return pl.pallas_call(
        body,
        grid=(M // blk,),
        in_specs=[sp, sp, sp,
                  pl.BlockSpec((2 * H, H), lambda i: (0, 0)),
                  pl.BlockSpec((H, H), lambda i: (0, 0)),
                  pl.BlockSpec((1, H), lambda i: (0, 0))],
        out_specs=sp,
        out_shape=_f32(M, H),
    )(s_rows, d_rows, e0, Wsd, We, b)


def _msg_step(s_rows, d_rows, e_prev, m_prev, Wsd, We, b, blk):
    M = e_prev.shape[0]

    def body(sr, dr, er, mr, wsdr, wer, br, enew, mout):
        e = er[...] + mr[...]
        enew[...] = e
        sd = jnp.concatenate([sr[...], dr[...]], axis=1)
        d1 = jnp.dot(sd, wsdr[...], preferred_element_type=jnp.float32)
        d3 = jnp.dot(e, wer[...], preferred_element_type=jnp.float32)
        mout[...] = jnp.maximum(d1 + d3 + br[...], 0.0)

    sp = pl.BlockSpec((blk, H), lambda i: (i, 0))
    return pl.pallas_call(
        body,
        grid=(M // blk,),
        in_specs=[sp, sp, sp, sp,
                  pl.BlockSpec((2 * H, H), lambda i: (0, 0)),
                  pl.BlockSpec((H, H), lambda i: (0, 0)),
                  pl.BlockSpec((1, H), lambda i: (0, 0))],
        out_specs=[sp, sp],
        out_shape=(_f32(M, H), _f32(M, H)),
    )(s_rows, d_rows, e_prev, m_prev, Wsd, We, b)


def _node_update(nodes, agg, Wu, bu, blk):
    def body(nr, ar, wr, br, outr):
        nb = nr[...]
        na = jnp.concatenate([nb, ar[...]], axis=1)
        upd = jnp.maximum(
            jnp.dot(na, wr[...], preferred_element_type=jnp.float32)
            + br[...], 0.0)
        outr[...] = nb + upd

    sp = pl.BlockSpec((blk, H), lambda i: (i, 0))
    return pl.pallas_call(
        body,
        grid=(N // blk,),
        in_specs=[sp, sp,
                  pl.BlockSpec((2 * H, H), lambda i: (0, 0)),
                  pl.BlockSpec((1, H), lambda i: (0, 0))],
        out_specs=sp,
        out_shape=_f32(N, H),
    )(nodes, agg, Wu, bu)


def _decoder(nodes, Wd8, bd8):
    def body(nr, wr, br, outr):
        outr[...] = jnp.dot(nr[...], wr[...],
                            preferred_element_type=jnp.float32) + br[...]

    return pl.pallas_call(
        body,
        grid=(5,),
        in_specs=[pl.BlockSpec((2000, H), lambda i: (i, 0)),
                  pl.BlockSpec((H, 8), lambda i: (0, 0)),
                  pl.BlockSpec((1, 8), lambda i: (0, 0))],
        out_specs=pl.BlockSpec((2000, 8), lambda i: (i, 0)),
        out_shape=_f32(N, 8),
    )(nodes, Wd8, bd8)


def _postproc(params, vai_a, vmi_a, vaj_a, vmj_a):
    rows = params[0].shape[0]
    has_tap = len(params) == 6
    nin = len(params) + 4

    def body(*refs):
        if has_tap:
            r, x, bfr, bto, tap, shift = (refs[k][...] for k in range(6))
        else:
            r, x, bfr, bto = (refs[k][...] for k in range(4))
        vai, vmi, vaj, vmj = (refs[len(params) + k][...] for k in range(4))
        o_refr, o_imfr, o_reto, o_imto = refs[nin:]
        den = r * r + x * x
        g = r / den
        by = -x / den
        if has_tap:
            th = vai - vaj - shift
            w = vmi * vmj / tap
            p2 = vmi * vmi / (tap * tap)
        else:
            th = vai - vaj
            w = vmi * vmj
            p2 = vmi * vmi
        c = jnp.cos(th)
        s = jnp.sin(th)
        q2 = vmj * vmj
        gc = g * c
        bys = by * s
        gs = g * s
        byc = by * c
        o_refr[...] = g * p2 - w * (gc + bys)
        o_imfr[...] = -(by + bfr) * p2 - w * (gs - byc)
        o_reto[...] = g * q2 - w * (gc - bys)
        o_imto[...] = -(by + bto) * q2 + w * (gs + byc)

    shp = _f32(rows, 128)
    return pl.pallas_call(
        body,
        out_shape=(shp, shp, shp, shp),
    )(*params, vai_a, vmi_a, vaj_a, vmj_a)


def _sc_rows_body(nodes, sac, dac, s_tr, d_tr,
                  sr_ac, dr_ac, sr_tr, dr_tr,
                  sidx0, didx0, sidx1, didx1, rA0, rB0, rA1, rB1,
                  nsh, semI0, semI1, semG0, semG1, semO0, semO1):
    c = lax.axis_index("c")
    s = lax.axis_index("s")
    wid = c * NS + s

    base0 = s * 624
    pltpu.sync_copy(nodes.at[pl.ds(base0, 624)], nsh.at[pl.ds(base0, 624)])

    @pl.when(s == NS - 1)
    def _stage_tail():
        pltpu.sync_copy(nodes.at[pl.ds(NS * 624, 16)],
                        nsh.at[pl.ds(NS * 624, 16)])
    plsc.subcore_barrier()

    def run_family(src, dst, outS, outD, per_w, ch, nch):
        sidx = (sidx0.at[pl.ds(0, ch)], sidx1.at[pl.ds(0, ch)])
        didx = (didx0.at[pl.ds(0, ch)], didx1.at[pl.ds(0, ch)])
        rA = (rA0.at[pl.ds(0, ch)], rA1.at[pl.ds(0, ch)])
        rB = (rB0.at[pl.ds(0, ch)], rB1.at[pl.ds(0, ch)])
        semI = (semI0, semI1)
        semG = (semG0, semG1)
        semO = (semO0, semO1)

        def fire_idx(j, k):
            base = wid * per_w + j * ch
            pltpu.async_copy(src.at[pl.ds(base, ch)], sidx[k], semI[k])
            pltpu.async_copy(dst.at[pl.ds(base, ch)], didx[k], semI[k])

        def drain_rows(sem, ref, n):
            for _ in range(n):
                pltpu.make_async_copy(outS.at[pl.ds(0, ch)], ref, sem).wait()

        def drain_idx(sem, ref, n):
            for _ in range(n):
                pltpu.make_async_copy(src.at[pl.ds(0, ch)], ref, sem).wait()

        def half(jj, a, k):
            @pl.when(jj > 0)
            def _():
                drain_rows(semO[k], rA[k], 2)
            drain_idx(semI[k], sidx[k], 2)
            pltpu.async_copy(nsh.at[sidx[k]], rA[k], semG[k])
            pltpu.async_copy(nsh.at[didx[k]], rB[k], semG[k])

        def finish(a, k):
            base = wid * per_w + a * ch
            drain_rows(semG[k], rA[k], 2)
            pltpu.async_copy(rA[k], outS.at[pl.ds(base, ch)], semO[k])
            pltpu.async_copy(rB[k], outD.at[pl.ds(base, ch)], semO[k])

            @pl.when(a + 2 < nch)
            def _():
                fire_idx(a + 2, k)

        fire_idx(0, 0)
        fire_idx(1, 1)

        def pair(jj, carry):
            a = 2 * jj
            half(jj, a, 0)
            half(jj, a + 1, 1)
            finish(a, 0)
            finish(a + 1, 1)
            return carry
        lax.fori_loop(0, nch // 2, pair, 0)

        t = nch - (nch % 2)
        if nch % 2:
            drain_rows(semO[0], rA[0], 2)
            drain_idx(semI[0], sidx[0], 2)
            pltpu.async_copy(nsh.at[sidx[0]], rA[0], semG[0])
            pltpu.async_copy(nsh.at[didx[0]], rB[0], semG[0])
            base = wid * per_w + t * ch
            drain_rows(semG[0], rA[0], 2)
            pltpu.async_copy(rA[0], outS.at[pl.ds(base, ch)], semO[0])
            pltpu.async_copy(rB[0], outD.at[pl.ds(base, ch)], semO[0])
        drain_rows(semO[0], rA[0], 2)
        drain_rows(semO[1], rA[1], 2)

    run_family(sac, dac, sr_ac, dr_ac, AC_PER_W, AC_CH, AC_NCH)
    run_family(s_tr, d_tr, sr_tr, dr_tr, TR_PER_W, TR_CH, TR_NCH)


@functools.cache
def _sc_rows_kernel():
    return pl.kernel(
        _sc_rows_body,
        out_type=(_f32(EAC, H), _f32(EAC, H), _f32(ETR, H), _f32(ETR, H)),
        mesh=_sc_mesh(),
        scratch_types=[
            pltpu.VMEM((AC_CH,), jnp.int32),
            pltpu.VMEM((AC_CH,), jnp.int32),
            pltpu.VMEM((AC_CH,), jnp.int32),
            pltpu.VMEM((AC_CH,), jnp.int32),
            pltpu.VMEM((AC_CH, H), jnp.float32),
            pltpu.VMEM((AC_CH, H), jnp.float32),
            pltpu.VMEM((AC_CH, H), jnp.float32),
            pltpu.VMEM((AC_CH, H), jnp.float32),
            pltpu.VMEM_SHARED((N, H), jnp.float32),
            pltpu.SemaphoreType.DMA,
            pltpu.SemaphoreType.DMA,
            pltpu.SemaphoreType.DMA,
            pltpu.SemaphoreType.DMA,
            pltpu.SemaphoreType.DMA,
            pltpu.SemaphoreType.DMA,
        ],
    )


def _sc_rows(*args):
    return _sc_rows_kernel()(*args)


def _sc_gather_body(tab, icat, jcat, vai_h, vmi_h, vaj_h, vmj_h,
                    tabv, sidx, didx, tsidx, tdidx, bai, bmi, baj, bmj):
    c = lax.axis_index("c")
    s = lax.axis_index("s")
    wid = c * NS + s

    pltpu.sync_copy(tab, tabv)
    zero16 = jnp.zeros((LANES,), jnp.int32)
    tsidx[pl.ds(32, LANES)] = zero16
    tdidx[pl.ds(32, LANES)] = zero16

    def gather16(bsidx, bdidx, k):
        sl = pl.ds(k * LANES, LANES)
        sr = bsidx[sl] * 8
        dr = bdidx[sl] * 8
        bai[sl] = plsc.load_gather(tabv, [sr])
        bmi[sl] = plsc.load_gather(tabv, [sr + 3])
        baj[sl] = plsc.load_gather(tabv, [dr])
        bmj[sl] = plsc.load_gather(tabv, [dr + 3])

    def ac_chunk(j, carry):
        base = wid * AC_PER_W + j * PP_AC_CH
        pltpu.sync_copy(icat.at[pl.ds(base, PP_AC_CH)], sidx)
        pltpu.sync_copy(jcat.at[pl.ds(base, PP_AC_CH)], didx)
        for k in range(PP_AC_CH // LANES):
            gather16(sidx, didx, k)
        pltpu.sync_copy(bai.at[pl.ds(0, PP_AC_CH)], vai_h.at[pl.ds(base, PP_AC_CH)])
        pltpu.sync_copy(bmi.at[pl.ds(0, PP_AC_CH)], vmi_h.at[pl.ds(base, PP_AC_CH)])
        pltpu.sync_copy(baj.at[pl.ds(0, PP_AC_CH)], vaj_h.at[pl.ds(base, PP_AC_CH)])
        pltpu.sync_copy(bmj.at[pl.ds(0, PP_AC_CH)], vmj_h.at[pl.ds(base, PP_AC_CH)])
        return carry
    lax.fori_loop(0, AC_PER_W // PP_AC_CH, ac_chunk, 0)

    def tr_chunk(j, carry):
        base = EAC + wid * TR_PER_W + j * PP_TR_CH
        pltpu.sync_copy(icat.at[pl.ds(base, PP_TR_CH)], tsidx.at[pl.ds(0, PP_TR_CH)])
        pltpu.sync_copy(jcat.at[pl.ds(base, PP_TR_CH)], tdidx.at[pl.ds(0, PP_TR_CH)])
        for k in range(3):
            gather16(tsidx, tdidx, k)
        pltpu.sync_copy(bai.at[pl.ds(0, PP_TR_CH)], vai_h.at[pl.ds(base, PP_TR_CH)])
        pltpu.sync_copy(bmi.at[pl.ds(0, PP_TR_CH)], vmi_h.at[pl.ds(base, PP_TR_CH)])
        pltpu.sync_copy(baj.at[pl.ds(0, PP_TR_CH)], vaj_h.at[pl.ds(base, PP_TR_CH)])
        pltpu.sync_copy(bmj.at[pl.ds(0, PP_TR_CH)], vmj_h.at[pl.ds(base, PP_TR_CH)])
        return carry
    lax.fori_loop(0, TR_PER_W // PP_TR_CH, tr_chunk, 0)


@functools.cache
def _sc_gather_kernel():
    return pl.kernel(
        _sc_gather_body,
        out_type=(_f32(ECAT), _f32(ECAT), _f32(ECAT), _f32(ECAT)),
        mesh=_sc_mesh(),
        compiler_params=pltpu.CompilerParams(needs_layout_passes=False),
        scratch_types=[
            pltpu.VMEM((N * 8,), jnp.float32),
            pltpu.VMEM((PP_AC_CH,), jnp.int32),
            pltpu.VMEM((PP_AC_CH,), jnp.int32),
            pltpu.VMEM((48,), jnp.int32),
            pltpu.VMEM((48,), jnp.int32),
            pltpu.VMEM((PP_AC_CH,), jnp.float32),
            pltpu.VMEM((PP_AC_CH,), jnp.float32),
            pltpu.VMEM((PP_AC_CH,), jnp.float32),
            pltpu.VMEM((PP_AC_CH,), jnp.float32),
        ],
    )


def _sc_gather(*args):
    return _sc_gather_kernel()(*args)


@jax.jit
def kernel(x_bus, edge_index_ac, edge_attr_ac, edge_index_tr, edge_attr_tr,
           W_enc_node, b_enc_node, W_enc_ac, b_enc_ac, W_enc_tr, b_enc_tr,
           W_msg_ac, b_msg_ac, W_msg_tr, b_msg_tr, W_upd, b_upd,
           W_dec, b_dec):
    sac = edge_index_ac[0]
    dac = edge_index_ac[1]
    s_tr = edge_index_tr[0]
    d_tr = edge_index_tr[1]

    xb8 = jnp.pad(x_bus, ((0, 0), (0, 4)))
    Wn8 = jnp.pad(W_enc_node, ((0, 4), (0, 0)))
    attr_ac16 = jnp.pad(edge_attr_ac, ((0, 0), (0, 7)))
    Wac16 = jnp.pad(W_enc_ac, ((0, 7), (0, 0)))
    attr_tr16 = jnp.pad(edge_attr_tr, ((0, 0), (0, 5)))
    Wtr16 = jnp.pad(W_enc_tr, ((0, 5), (0, 0)))
    Wd8 = jnp.pad(W_dec, ((0, 0), (0, 4)))
    bd8 = jnp.pad(b_dec, (0, 4)).reshape(1, 8)

    nodes = _mlp_relu(xb8, Wn8, b_enc_node.reshape(1, H), 2000)
    e_ac = _mlp_relu(attr_ac16, Wac16, b_enc_ac.reshape(1, H), 8000)
    e_tr = _mlp_relu(attr_tr16, Wtr16, b_enc_tr.reshape(1, H), 8000)

    m_ac = None
    m_tr = None
    for l in range(K_STEPS):
        Wsd_ac = W_msg_ac[l][:2 * H]
        We_ac = W_msg_ac[l][2 * H:]
        Wsd_tr = W_msg_tr[l][:2 * H]
        We_tr = W_msg_tr[l][2 * H:]
        bm_ac = b_msg_ac[l].reshape(1, H)
        bm_tr = b_msg_tr[l].reshape(1, H)

        sr_ac, dr_ac, sr_tr, dr_tr = _sc_rows(nodes, sac, dac, s_tr, d_tr)

        if l == 0:
            m_ac = _msg_first(sr_ac, dr_ac, e_ac, Wsd_ac, We_ac, bm_ac, 8000)
            m_tr = _msg_first(sr_tr, dr_tr, e_tr, Wsd_tr, We_tr, bm_tr, 8000)
        else:
            e_ac, m_ac = _msg_step(sr_ac, dr_ac, e_ac, m_ac,
                                   Wsd_ac, We_ac, bm_ac, 8000)
            e_tr, m_tr = _msg_step(sr_tr, dr_tr, e_tr, m_tr,
                                   Wsd_tr, We_tr, bm_tr, 8000)

        agg = jnp.zeros((N, H), jnp.float32).at[dac].add(m_ac)
        agg = agg.at[d_tr].add(m_tr)

        nodes = _node_update(nodes, agg, W_upd[l], b_upd[l].reshape(1, H),
                             2000)

    out8 = _decoder(nodes, Wd8, bd8)
    out_bus = out8[:, :4]

    icat = jnp.concatenate([sac, s_tr])
    jcat = jnp.concatenate([dac, d_tr])
    vai, vmi, vaj, vmj = _sc_gather(out8.reshape(-1), icat, jcat)

    def _rs(v, lo, hi):
        return v[lo:hi].reshape(-1, 128)

    ac_params = tuple(_rs(edge_attr_ac[:, k], 0, EAC) for k in (4, 5, 2, 3))
    tr_params = tuple(_rs(edge_attr_tr[:, k], 0, ETR)
                      for k in (2, 3, 9, 10, 7, 8))
    pp_ac = _postproc(ac_params, _rs(vai, 0, EAC), _rs(vmi, 0, EAC),
                      _rs(vaj, 0, EAC), _rs(vmj, 0, EAC))
    pp_tr = _postproc(tr_params, _rs(vai, EAC, ECAT), _rs(vmi, EAC, ECAT),
                      _rs(vaj, EAC, ECAT), _rs(vmj, EAC, ECAT))

    re_fr = jnp.concatenate([pp_ac[0].reshape(-1), pp_tr[0].reshape(-1)])
    im_fr = jnp.concatenate([pp_ac[1].reshape(-1), pp_tr[1].reshape(-1)])
    re_to = jnp.concatenate([pp_ac[2].reshape(-1), pp_tr[2].reshape(-1)])
    im_to = jnp.concatenate([pp_ac[3].reshape(-1), pp_tr[3].reshape(-1)])
    return (out_bus, re_fr, im_fr, re_to, im_to)


# speedup vs baseline: 3.2584x; 1.0138x over previous
import functools

import jax
import jax.numpy as jnp
from jax import lax
from jax.experimental import pallas as pl
from jax.experimental.pallas import tpu as pltpu
from jax.experimental.pallas import tpu_sc as plsc

N = 10000
EAC = 320000
ETR = 32000
H = 128
K_STEPS = 4
LANES = 16
NC, NS = 2, 16
NW = NC * NS

AC_PER_W = EAC // NW
TR_PER_W = ETR // NW
AC_CH = 80
TR_CH = 40
AC_NCH = AC_PER_W // AC_CH
TR_NCH = TR_PER_W // TR_CH

ECAT = EAC + ETR
PP_AC_CH = 80
PP_TR_CH = 40


@functools.cache
def _sc_mesh():
    return plsc.VectorSubcoreMesh(core_axis_name="c", subcore_axis_name="s",
                                  num_cores=NC, num_subcores=NS)


def _f32(*shape):
    return jax.ShapeDtypeStruct(shape, jnp.float32)


def _mlp_relu(x, W, b, blk):
    M, K = x.shape
    Hd = W.shape[1]

    def body(xr, wr, br, outr):
        outr[...] = jnp.maximum(
            jnp.dot(xr[...], wr[...], preferred_element_type=jnp.float32)
            + br[...], 0.0)

    return pl.pallas_call(
        body,
        grid=(M // blk,),
        in_specs=[pl.BlockSpec((blk, K), lambda i: (i, 0)),
                  pl.BlockSpec((K, Hd), lambda i: (0, 0)),
                  pl.BlockSpec((1, Hd), lambda i: (0, 0))],
        out_specs=pl.BlockSpec((blk, Hd), lambda i: (i, 0)),
        out_shape=_f32(M, Hd),
    )(x, W, b)


def _msg_first(s_rows, d_rows, e0, Wsd, We, b, blk):
    M = e0.shape[0]

    def body(sr, dr, er, wsdr, wer, br, mout):
        sd = jnp.concatenate([sr[...], dr[...]], axis=1)
        d1 = jnp.dot(sd, wsdr[...], preferred_element_type=jnp.float32)
        d3 = jnp.dot(er[...], wer[...], preferred_element_type=jnp.float32)
        mout[...] = jnp.maximum(d1 + d3 + br[...], 0.0)

    sp = pl.BlockSpec((blk, H), lambda i: (i, 0))
    return pl.pallas_call(
        body,
        grid=(M // blk,),
        in_specs=[sp, sp, sp,
                  pl.BlockSpec((2 * H, H), lambda i: (0, 0)),
                  pl.BlockSpec((H, H), lambda i: (0, 0)),
                  pl.BlockSpec((1, H), lambda i: (0, 0))],
        out_specs=sp,
        out_shape=_f32(M, H),
    )(s_rows, d_rows, e0, Wsd, We, b)


def _e_add(e, m, blk):
    M = e.shape[0]

    def body(er, mr, outr):
        outr[...] = er[...] + mr[...]

    sp = pl.BlockSpec((blk, H), lambda i: (i, 0))
    return pl.pallas_call(
        body,
        grid=(M // blk,),
        in_specs=[sp, sp],
        out_specs=sp,
        out_shape=_f32(M, H),
    )(e, m)


def _node_update(nodes, agg, Wu, bu, blk):
    def body(nr, ar, wr, br, outr):
        nb = nr[...]
        na = jnp.concatenate([nb, ar[...]], axis=1)
        upd = jnp.maximum(
            jnp.dot(na, wr[...], preferred_element_type=jnp.float32)
            + br[...], 0.0)
        outr[...] = nb + upd

    sp = pl.BlockSpec((blk, H), lambda i: (i, 0))
    return pl.pallas_call(
        body,
        grid=(N // blk,),
        in_specs=[sp, sp,
                  pl.BlockSpec((2 * H, H), lambda i: (0, 0)),
                  pl.BlockSpec((1, H), lambda i: (0, 0))],
        out_specs=sp,
        out_shape=_f32(N, H),
    )(nodes, agg, Wu, bu)


def _decoder(nodes, Wd8, bd8):
    def body(nr, wr, br, outr):
        outr[...] = jnp.dot(nr[...], wr[...],
                            preferred_element_type=jnp.float32) + br[...]

    return pl.pallas_call(
        body,
        grid=(5,),
        in_specs=[pl.BlockSpec((2000, H), lambda i: (i, 0)),
                  pl.BlockSpec((H, 8), lambda i: (0, 0)),
                  pl.BlockSpec((1, 8), lambda i: (0, 0))],
        out_specs=pl.BlockSpec((2000, 8), lambda i: (i, 0)),
        out_shape=_f32(N, 8),
    )(nodes, Wd8, bd8)


def _postproc(params, vai_a, vmi_a, vaj_a, vmj_a):
    rows = params[0].shape[0]
    has_tap = len(params) == 6
    nin = len(params) + 4

    def body(*refs):
        if has_tap:
            r, x, bfr, bto, tap, shift = (refs[k][...] for k in range(6))
        else:
            r, x, bfr, bto = (refs[k][...] for k in range(4))
        vai, vmi, vaj, vmj = (refs[len(params) + k][...] for k in range(4))
        o_refr, o_imfr, o_reto, o_imto = refs[nin:]
        den = r * r + x * x
        g = r / den
        by = -x / den
        if has_tap:
            th = vai - vaj - shift
            w = vmi * vmj / tap
            p2 = vmi * vmi / (tap * tap)
        else:
            th = vai - vaj
            w = vmi * vmj
            p2 = vmi * vmi
        c = jnp.cos(th)
        s = jnp.sin(th)
        q2 = vmj * vmj
        gc = g * c
        bys = by * s
        gs = g * s
        byc = by * c
        o_refr[...] = g * p2 - w * (gc + bys)
        o_imfr[...] = -(by + bfr) * p2 - w * (gs - byc)
        o_reto[...] = g * q2 - w * (gc - bys)
        o_imto[...] = -(by + bto) * q2 + w * (gs + byc)

    shp = _f32(rows, 128)
    return pl.pallas_call(
        body,
        out_shape=(shp, shp, shp, shp),
    )(*params, vai_a, vmi_a, vaj_a, vmj_a)


def _sc_rows_body(nodes, sac, dac, s_tr, d_tr,
                  sr_ac, dr_ac, sr_tr, dr_tr,
                  sidx0, didx0, sidx1, didx1, rA0, rB0, rA1, rB1,
                  nsh, semI0, semI1, semG0, semG1, semO0, semO1):
    c = lax.axis_index("c")
    s = lax.axis_index("s")
    wid = c * NS + s

    base0 = s * 624
    pltpu.sync_copy(nodes.at[pl.ds(base0, 624)], nsh.at[pl.ds(base0, 624)])

    @pl.when(s == NS - 1)
    def _stage_tail():
        pltpu.sync_copy(nodes.at[pl.ds(NS * 624, 16)],
                        nsh.at[pl.ds(NS * 624, 16)])
    plsc.subcore_barrier()

    def run_family(src, dst, outS, outD, per_w, ch, nch):
        sidx = (sidx0.at[pl.ds(0, ch)], sidx1.at[pl.ds(0, ch)])
        didx = (didx0.at[pl.ds(0, ch)], didx1.at[pl.ds(0, ch)])
        rA = (rA0.at[pl.ds(0, ch)], rA1.at[pl.ds(0, ch)])
        rB = (rB0.at[pl.ds(0, ch)], rB1.at[pl.ds(0, ch)])
        semI = (semI0, semI1)
        semG = (semG0, semG1)
        semO = (semO0, semO1)

        def fire_idx(j, k):
            base = wid * per_w + j * ch
            pltpu.async_copy(src.at[pl.ds(base, ch)], sidx[k], semI[k])
            pltpu.async_copy(dst.at[pl.ds(base, ch)], didx[k], semI[k])

        def drain_rows(sem, ref, n):
            for _ in range(n):
                pltpu.make_async_copy(outS.at[pl.ds(0, ch)], ref, sem).wait()

        def drain_idx(sem, ref, n):
            for _ in range(n):
                pltpu.make_async_copy(src.at[pl.ds(0, ch)], ref, sem).wait()

        def half(jj, a, k):
            @pl.when(jj > 0)
            def _():
                drain_rows(semO[k], rA[k], 2)
            drain_idx(semI[k], sidx[k], 2)
            pltpu.async_copy(nsh.at[sidx[k]], rA[k], semG[k])
            pltpu.async_copy(nsh.at[didx[k]], rB[k], semG[k])

        def finish(a, k):
            base = wid * per_w + a * ch
            drain_rows(semG[k], rA[k], 2)
            pltpu.async_copy(rA[k], outS.at[pl.ds(base, ch)], semO[k])
            pltpu.async_copy(rB[k], outD.at[pl.ds(base, ch)], semO[k])

            @pl.when(a + 2 < nch)
            def _():
                fire_idx(a + 2, k)

        fire_idx(0, 0)
        fire_idx(1, 1)

        def pair(jj, carry):
            a = 2 * jj
            half(jj, a, 0)
            half(jj, a + 1, 1)
            finish(a, 0)
            finish(a + 1, 1)
            return carry
        lax.fori_loop(0, nch // 2, pair, 0)

        t = nch - (nch % 2)
        if nch % 2:
            drain_rows(semO[0], rA[0], 2)
            drain_idx(semI[0], sidx[0], 2)
            pltpu.async_copy(nsh.at[sidx[0]], rA[0], semG[0])
            pltpu.async_copy(nsh.at[didx[0]], rB[0], semG[0])
            base = wid * per_w + t * ch
            drain_rows(semG[0], rA[0], 2)
            pltpu.async_copy(rA[0], outS.at[pl.ds(base, ch)], semO[0])
            pltpu.async_copy(rB[0], outD.at[pl.ds(base, ch)], semO[0])
        drain_rows(semO[0], rA[0], 2)
        drain_rows(semO[1], rA[1], 2)

    run_family(sac, dac, sr_ac, dr_ac, AC_PER_W, AC_CH, AC_NCH)
    run_family(s_tr, d_tr, sr_tr, dr_tr, TR_PER_W, TR_CH, TR_NCH)


@functools.cache
def _sc_rows_kernel():
    return pl.kernel(
        _sc_rows_body,
        out_type=(_f32(EAC, H), _f32(EAC, H), _f32(ETR, H), _f32(ETR, H)),
        mesh=_sc_mesh(),
        scratch_types=[
            pltpu.VMEM((AC_CH,), jnp.int32),
            pltpu.VMEM((AC_CH,), jnp.int32),
            pltpu.VMEM((AC_CH,), jnp.int32),
            pltpu.VMEM((AC_CH,), jnp.int32),
            pltpu.VMEM((AC_CH, H), jnp.float32),
            pltpu.VMEM((AC_CH, H), jnp.float32),
            pltpu.VMEM((AC_CH, H), jnp.float32),
            pltpu.VMEM((AC_CH, H), jnp.float32),
            pltpu.VMEM_SHARED((N, H), jnp.float32),
            pltpu.SemaphoreType.DMA,
            pltpu.SemaphoreType.DMA,
            pltpu.SemaphoreType.DMA,
            pltpu.SemaphoreType.DMA,
            pltpu.SemaphoreType.DMA,
            pltpu.SemaphoreType.DMA,
        ],
    )


def _sc_rows(*args):
    return _sc_rows_kernel()(*args)


def _sc_gather_body(tab, icat, jcat, vai_h, vmi_h, vaj_h, vmj_h,
                    tabv, sidx, didx, tsidx, tdidx, bai, bmi, baj, bmj):
    c = lax.axis_index("c")
    s = lax.axis_index("s")
    wid = c * NS + s

    pltpu.sync_copy(tab, tabv)
    zero16 = jnp.zeros((LANES,), jnp.int32)
    tsidx[pl.ds(32, LANES)] = zero16
    tdidx[pl.ds(32, LANES)] = zero16

    def gather16(bsidx, bdidx, k):
        sl = pl.ds(k * LANES, LANES)
        sr = bsidx[sl] * 8
        dr = bdidx[sl] * 8
        bai[sl] = plsc.load_gather(tabv, [sr])
        bmi[sl] = plsc.load_gather(tabv, [sr + 3])
        baj[sl] = plsc.load_gather(tabv, [dr])
        bmj[sl] = plsc.load_gather(tabv, [dr + 3])

    def ac_chunk(j, carry):
        base = wid * AC_PER_W + j * PP_AC_CH
        pltpu.sync_copy(icat.at[pl.ds(base, PP_AC_CH)], sidx)
        pltpu.sync_copy(jcat.at[pl.ds(base, PP_AC_CH)], didx)
        for k in range(PP_AC_CH // LANES):
            gather16(sidx, didx, k)
        pltpu.sync_copy(bai.at[pl.ds(0, PP_AC_CH)], vai_h.at[pl.ds(base, PP_AC_CH)])
        pltpu.sync_copy(bmi.at[pl.ds(0, PP_AC_CH)], vmi_h.at[pl.ds(base, PP_AC_CH)])
        pltpu.sync_copy(baj.at[pl.ds(0, PP_AC_CH)], vaj_h.at[pl.ds(base, PP_AC_CH)])
        pltpu.sync_copy(bmj.at[pl.ds(0, PP_AC_CH)], vmj_h.at[pl.ds(base, PP_AC_CH)])
        return carry
    lax.fori_loop(0, AC_PER_W // PP_AC_CH, ac_chunk, 0)

    def tr_chunk(j, carry):
        base = EAC + wid * TR_PER_W + j * PP_TR_CH
        pltpu.sync_copy(icat.at[pl.ds(base, PP_TR_CH)], tsidx.at[pl.ds(0, PP_TR_CH)])
        pltpu.sync_copy(jcat.at[pl.ds(base, PP_TR_CH)], tdidx.at[pl.ds(0, PP_TR_CH)])
        for k in range(3):
            gather16(tsidx, tdidx, k)
        pltpu.sync_copy(bai.at[pl.ds(0, PP_TR_CH)], vai_h.at[pl.ds(base, PP_TR_CH)])
        pltpu.sync_copy(bmi.at[pl.ds(0, PP_TR_CH)], vmi_h.at[pl.ds(base, PP_TR_CH)])
        pltpu.sync_copy(baj.at[pl.ds(0, PP_TR_CH)], vaj_h.at[pl.ds(base, PP_TR_CH)])
        pltpu.sync_copy(bmj.at[pl.ds(0, PP_TR_CH)], vmj_h.at[pl.ds(base, PP_TR_CH)])
        return carry
    lax.fori_loop(0, TR_PER_W // PP_TR_CH, tr_chunk, 0)


@functools.cache
def _sc_gather_kernel():
    return pl.kernel(
        _sc_gather_body,
        out_type=(_f32(ECAT), _f32(ECAT), _f32(ECAT), _f32(ECAT)),
        mesh=_sc_mesh(),
        compiler_params=pltpu.CompilerParams(needs_layout_passes=False),
        scratch_types=[
            pltpu.VMEM((N * 8,), jnp.float32),
            pltpu.VMEM((PP_AC_CH,), jnp.int32),
            pltpu.VMEM((PP_AC_CH,), jnp.int32),
            pltpu.VMEM((48,), jnp.int32),
            pltpu.VMEM((48,), jnp.int32),
            pltpu.VMEM((PP_AC_CH,), jnp.float32),
            pltpu.VMEM((PP_AC_CH,), jnp.float32),
            pltpu.VMEM((PP_AC_CH,), jnp.float32),
            pltpu.VMEM((PP_AC_CH,), jnp.float32),
        ],
    )


def _sc_gather(*args):
    return _sc_gather_kernel()(*args)


@jax.jit
def kernel(x_bus, edge_index_ac, edge_attr_ac, edge_index_tr, edge_attr_tr,
           W_enc_node, b_enc_node, W_enc_ac, b_enc_ac, W_enc_tr, b_enc_tr,
           W_msg_ac, b_msg_ac, W_msg_tr, b_msg_tr, W_upd, b_upd,
           W_dec, b_dec):
    sac = edge_index_ac[0]
    dac = edge_index_ac[1]
    s_tr = edge_index_tr[0]
    d_tr = edge_index_tr[1]

    xb8 = jnp.pad(x_bus, ((0, 0), (0, 4)))
    Wn8 = jnp.pad(W_enc_node, ((0, 4), (0, 0)))
    attr_ac16 = jnp.pad(edge_attr_ac, ((0, 0), (0, 7)))
    Wac16 = jnp.pad(W_enc_ac, ((0, 7), (0, 0)))
    attr_tr16 = jnp.pad(edge_attr_tr, ((0, 0), (0, 5)))
    Wtr16 = jnp.pad(W_enc_tr, ((0, 5), (0, 0)))
    Wd8 = jnp.pad(W_dec, ((0, 0), (0, 4)))
    bd8 = jnp.pad(b_dec, (0, 4)).reshape(1, 8)

    nodes = _mlp_relu(xb8, Wn8, b_enc_node.reshape(1, H), 2000)
    e_ac = _mlp_relu(attr_ac16, Wac16, b_enc_ac.reshape(1, H), 8000)
    e_tr = _mlp_relu(attr_tr16, Wtr16, b_enc_tr.reshape(1, H), 8000)

    m_ac = None
    m_tr = None
    for l in range(K_STEPS):
        Wsd_ac = W_msg_ac[l][:2 * H]
        We_ac = W_msg_ac[l][2 * H:]
        Wsd_tr = W_msg_tr[l][:2 * H]
        We_tr = W_msg_tr[l][2 * H:]
        bm_ac = b_msg_ac[l].reshape(1, H)
        bm_tr = b_msg_tr[l].reshape(1, H)

        if l > 0:
            e_ac = _e_add(e_ac, m_ac, 8000)
            e_tr = _e_add(e_tr, m_tr, 8000)

        sr_ac, dr_ac, sr_tr, dr_tr = _sc_rows(nodes, sac, dac, s_tr, d_tr)

        m_ac = _msg_first(sr_ac, dr_ac, e_ac, Wsd_ac, We_ac, bm_ac, 8000)
        m_tr = _msg_first(sr_tr, dr_tr, e_tr, Wsd_tr, We_tr, bm_tr, 8000)

        agg = jnp.zeros((N, H), jnp.float32).at[dac].add(m_ac)
        agg = agg.at[d_tr].add(m_tr)

        nodes = _node_update(nodes, agg, W_upd[l], b_upd[l].reshape(1, H),
                             2000)

    out8 = _decoder(nodes, Wd8, bd8)
    out_bus = out8[:, :4]

    icat = jnp.concatenate([sac, s_tr])
    jcat = jnp.concatenate([dac, d_tr])
    vai, vmi, vaj, vmj = _sc_gather(out8.reshape(-1), icat, jcat)

    def _rs(v, lo, hi):
        return v[lo:hi].reshape(-1, 128)

    ac_params = tuple(_rs(edge_attr_ac[:, k], 0, EAC) for k in (4, 5, 2, 3))
    tr_params = tuple(_rs(edge_attr_tr[:, k], 0, ETR)
                      for k in (2, 3, 9, 10, 7, 8))
    pp_ac = _postproc(ac_params, _rs(vai, 0, EAC), _rs(vmi, 0, EAC),
                      _rs(vaj, 0, EAC), _rs(vmj, 0, EAC))
    pp_tr = _postproc(tr_params, _rs(vai, EAC, ECAT), _rs(vmi, EAC, ECAT),
                      _rs(vaj, EAC, ECAT), _rs(vmj, EAC, ECAT))

    re_fr = jnp.concatenate([pp_ac[0].reshape(-1), pp_tr[0].reshape(-1)])
    im_fr = jnp.concatenate([pp_ac[1].reshape(-1), pp_tr[1].reshape(-1)])
    re_to = jnp.concatenate([pp_ac[2].reshape(-1), pp_tr[2].reshape(-1)])
    im_to = jnp.concatenate([pp_ac[3].reshape(-1), pp_tr[3].reshape(-1)])
    return (out_bus, re_fr, im_fr, re_to, im_to)
